# baseline (speedup 1.0000x reference)
import functools

import jax
import jax.numpy as jnp
from jax import lax
from jax.experimental import pallas as pl
from jax.experimental.pallas import tpu as pltpu
from jax.experimental.pallas import tpu_sc as plsc

N = 10000
E = 320000
D = 128

NC = 2
NS = 16
NW = NC * NS

CHUNK = 128
NCHUNKS = E // CHUNK
CHUNKS_PER_W = NCHUNKS // NW
REM = NCHUNKS - CHUNKS_PER_W * NW

RPS = 632
RPS_LAST = N - RPS * (NS - 1)


def _sc_segment_sum(feat, src, dst):
    mesh = plsc.VectorSubcoreMesh(core_axis_name="c", subcore_axis_name="s")

    @functools.partial(
        pl.kernel,
        out_type=jax.ShapeDtypeStruct((NC * N, D), jnp.float32),
        mesh=mesh,
        scratch_types=(
            [pltpu.VMEM((CHUNK,), jnp.int32)] * 8 +
            [
                pltpu.VMEM((CHUNK, D), jnp.float32),
                pltpu.VMEM((CHUNK, D), jnp.float32),
                pltpu.VMEM_SHARED((N, D), jnp.float32),
            ] +
            [pltpu.SemaphoreType.DMA] * 8
        ),
    )
    def k(feat_hbm, src_hbm, dst_hbm, out_hbm,
          sidx0, sidx1, sidx2, sidx3, didx0, didx1, didx2, didx3,
          rows0, rows1, acc, i0, i1, i2, i3, g0, g1, s0, s1):
        c = lax.axis_index("c")
        s = lax.axis_index("s")
        wid = c * NS + s

        @pl.loop(0, CHUNK)
        def _(i):
            @pl.loop(0, D, step=16)
            def _(j):
                rows0.at[i, pl.ds(j, 16)][...] = jnp.zeros((16,), jnp.float32)

        base_r = s * RPS

        def zero_rows(tail):
            @pl.loop(0, 4)
            def _(r):
                pltpu.sync_copy(rows0, acc.at[pl.ds(base_r + r * CHUNK, CHUNK)])
            pltpu.sync_copy(rows0.at[pl.ds(0, tail)],
                            acc.at[pl.ds(base_r + 4 * CHUNK, tail)])

        @pl.when(s < NS - 1)
        def _():
            zero_rows(RPS - 4 * CHUNK)

        @pl.when(s == NS - 1)
        def _():
            zero_rows(RPS_LAST - 4 * CHUNK)

        plsc.subcore_barrier()

        base_c = wid * CHUNKS_PER_W
        SIDX = (sidx0, sidx1, sidx2, sidx3)
        DIDX = (didx0, didx1, didx2, didx3)
        ISEM = (i0, i1, i2, i3)
        ROWS = (rows0, rows1)
        GSEM = (g0, g1)
        SSEM = (s0, s1)

        def idx_start(u, q):
            e0 = (base_c + u) * CHUNK
            pltpu.async_copy(src_hbm.at[pl.ds(e0, CHUNK)], SIDX[q], ISEM[q])
            pltpu.async_copy(dst_hbm.at[pl.ds(e0, CHUNK)], DIDX[q], ISEM[q])

        def idx_wait(u, q):
            e0 = (base_c + u) * CHUNK
            pltpu.make_async_copy(src_hbm.at[pl.ds(e0, CHUNK)], SIDX[q],
                                  ISEM[q]).wait()
            pltpu.make_async_copy(dst_hbm.at[pl.ds(e0, CHUNK)], DIDX[q],
                                  ISEM[q]).wait()

        def gather_start(q, r):
            pltpu.async_copy(feat_hbm.at[SIDX[q]], ROWS[r], GSEM[r])

        def gather_wait(q, r):
            pltpu.make_async_copy(feat_hbm.at[SIDX[q]], ROWS[r],
                                  GSEM[r]).wait()

        def scatter_start(q, r):
            pltpu.async_copy(ROWS[r], acc.at[DIDX[q]], SSEM[r], add=True)

        def scatter_wait(q, r):
            pltpu.make_async_copy(ROWS[r], acc.at[DIDX[q]], SSEM[r]).wait()

        idx_start(0, 0)
        idx_start(1, 1)
        idx_wait(0, 0)
        gather_start(0, 0)
        idx_wait(1, 1)
        gather_start(1, 1)
        idx_start(2, 2)
        gather_wait(0, 0)
        scatter_start(0, 0)
        scatter_wait(0, 0)
        idx_wait(2, 2)
        gather_start(2, 0)
        idx_start(3, 3)
        gather_wait(1, 1)
        scatter_start(1, 1)

        @pl.loop(2, CHUNKS_PER_W, step=4)
        def _(t):
            for st in range(4):
                q = (2 + st) % 4
                r = (2 + st) % 2
                qn = (q + 1) % 4
                qp = (q + 2) % 4
                qm = (q + 3) % 4
                rn = (r + 1) % 2
                u = t + st
                scatter_wait(qm, rn)
                idx_wait(u + 1, qn)
                gather_start(qn, rn)
                idx_start(u + 2, qp)
                gather_wait(q, r)
                scatter_start(q, r)

        scatter_wait(1, 1)
        gather_wait(2, 0)
        idx_wait(CHUNKS_PER_W + 1, 3)

        @pl.when(wid < REM // 2)
        def _():
            e0 = (NW * CHUNKS_PER_W + wid * 2) * CHUNK
            pltpu.sync_copy(src_hbm.at[pl.ds(e0, CHUNK)], sidx0)
            pltpu.sync_copy(dst_hbm.at[pl.ds(e0, CHUNK)], didx0)
            d0 = pltpu.async_copy(feat_hbm.at[sidx0], rows0, g0)
            pltpu.sync_copy(src_hbm.at[pl.ds(e0 + CHUNK, CHUNK)], sidx1)
            pltpu.sync_copy(dst_hbm.at[pl.ds(e0 + CHUNK, CHUNK)], didx1)
            d1 = pltpu.async_copy(feat_hbm.at[sidx1], rows1, g1)
            d0.wait()
            pltpu.sync_copy(rows0, acc.at[didx0], add=True)
            d1.wait()
            pltpu.sync_copy(rows1, acc.at[didx1], add=True)

        plsc.subcore_barrier()

        @pl.when(s < NS - 1)
        def _():
            pltpu.sync_copy(acc.at[pl.ds(base_r, RPS)],
                            out_hbm.at[pl.ds(c * N + base_r, RPS)])

        @pl.when(s == NS - 1)
        def _():
            pltpu.sync_copy(acc.at[pl.ds(base_r, RPS_LAST)],
                            out_hbm.at[pl.ds(c * N + base_r, RPS_LAST)])

    return k(feat, src, dst)


BR = 1000


def _tc_matmul(x, W, b2d):
    def body(x_ref, w_ref, b_ref, o_ref):
        o_ref[...] = jnp.dot(
            x_ref[...], w_ref[...], preferred_element_type=jnp.float32,
            precision=lax.Precision.HIGHEST) + b_ref[...]

    return pl.pallas_call(
        body,
        grid=(N // BR,),
        in_specs=[
            pl.BlockSpec((BR, D), lambda i: (i, 0)),
            pl.BlockSpec((D, D), lambda i: (0, 0)),
            pl.BlockSpec((1, D), lambda i: (0, 0)),
        ],
        out_specs=pl.BlockSpec((BR, D), lambda i: (i, 0)),
        out_shape=jax.ShapeDtypeStruct((N, D), jnp.float32),
    )(x, W, b2d)


def _tc_combine(z, p0, p1, W, final):
    def body(z_ref, p0_ref, p1_ref, w_ref, o_ref):
        t = p0_ref[...] + p1_ref[...]
        acc = jnp.dot(t, w_ref[...], preferred_element_type=jnp.float32,
                      precision=lax.Precision.HIGHEST) + z_ref[...]
        if final:
            m = jnp.max(acc, axis=1, keepdims=True)
            e = acc - m
            lse = jnp.log(jnp.sum(jnp.exp(e), axis=1, keepdims=True))
            o_ref[...] = e - lse
        else:
            o_ref[...] = jnp.maximum(acc, 0.0)

    return pl.pallas_call(
        body,
        grid=(N // BR,),
        in_specs=[
            pl.BlockSpec((BR, D), lambda i: (i, 0)),
            pl.BlockSpec((BR, D), lambda i: (i, 0)),
            pl.BlockSpec((BR, D), lambda i: (i, 0)),
            pl.BlockSpec((D, D), lambda i: (0, 0)),
        ],
        out_specs=pl.BlockSpec((BR, D), lambda i: (i, 0)),
        out_shape=jax.ShapeDtypeStruct((N, D), jnp.float32),
    )(z, p0, p1, W)


def kernel(input_feature, edge_index, W1, b1, W2, b2):
    src = edge_index[0]
    dst = edge_index[1]
    b1_2d = b1.reshape(1, D)
    b2_2d = b2.reshape(1, D)

    p = _sc_segment_sum(input_feature, src, dst)
    z1 = _tc_matmul(input_feature, W1, b1_2d)
    h = _tc_combine(z1, p[:N], p[N:], W1, final=False)
    q = _sc_segment_sum(h, src, dst)
    z2 = _tc_matmul(h, W2, b2_2d)
    return _tc_combine(z2, q[:N], q[N:], W2, final=True)


# speedup vs baseline: 1.0910x; 1.0910x over previous
import functools

import jax
import jax.numpy as jnp
from jax import lax
from jax.experimental import pallas as pl
from jax.experimental.pallas import tpu as pltpu
from jax.experimental.pallas import tpu_sc as plsc

N = 10000
E = 320000
D = 128

NC = 2
NS = 16
NW = NC * NS

CHUNK = 128
NCHUNKS = E // CHUNK
CHUNKS_PER_W = NCHUNKS // NW
REM = NCHUNKS - CHUNKS_PER_W * NW

RPS = 632
RPS_LAST = N - RPS * (NS - 1)


def _sc_segment_sum(feat, ei_flat):
    mesh = plsc.VectorSubcoreMesh(core_axis_name="c", subcore_axis_name="s")

    @functools.partial(
        pl.kernel,
        out_type=jax.ShapeDtypeStruct((NC * N, D), jnp.float32),
        mesh=mesh,
        scratch_types=(
            [pltpu.VMEM((CHUNK,), jnp.int32)] * 8 +
            [
                pltpu.VMEM((CHUNK, D), jnp.float32),
                pltpu.VMEM((CHUNK, D), jnp.float32),
                pltpu.VMEM_SHARED((N, D), jnp.float32),
            ] +
            [pltpu.SemaphoreType.DMA] * 8
        ),
    )
    def k(feat_hbm, ei_hbm, out_hbm,
          sidx0, sidx1, sidx2, sidx3, didx0, didx1, didx2, didx3,
          rows0, rows1, acc, i0, i1, i2, i3, g0, g1, s0, s1):
        c = lax.axis_index("c")
        s = lax.axis_index("s")
        wid = c * NS + s

        @pl.loop(0, CHUNK)
        def _(i):
            @pl.loop(0, D, step=16)
            def _(j):
                rows0.at[i, pl.ds(j, 16)][...] = jnp.zeros((16,), jnp.float32)

        base_r = s * RPS

        def zero_rows(tail):
            @pl.loop(0, 4)
            def _(r):
                pltpu.sync_copy(rows0, acc.at[pl.ds(base_r + r * CHUNK, CHUNK)])
            pltpu.sync_copy(rows0.at[pl.ds(0, tail)],
                            acc.at[pl.ds(base_r + 4 * CHUNK, tail)])

        @pl.when(s < NS - 1)
        def _():
            zero_rows(RPS - 4 * CHUNK)

        @pl.when(s == NS - 1)
        def _():
            zero_rows(RPS_LAST - 4 * CHUNK)

        plsc.subcore_barrier()

        base_c = wid * CHUNKS_PER_W
        SIDX = (sidx0, sidx1, sidx2, sidx3)
        DIDX = (didx0, didx1, didx2, didx3)
        ISEM = (i0, i1, i2, i3)
        ROWS = (rows0, rows1)
        GSEM = (g0, g1)
        SSEM = (s0, s1)

        def idx_start(u, q):
            e0 = (base_c + u) * CHUNK
            pltpu.async_copy(ei_hbm.at[pl.ds(e0, CHUNK)], SIDX[q], ISEM[q])
            pltpu.async_copy(ei_hbm.at[pl.ds(E + e0, CHUNK)], DIDX[q], ISEM[q])

        def idx_wait(u, q):
            e0 = (base_c + u) * CHUNK
            pltpu.make_async_copy(ei_hbm.at[pl.ds(e0, CHUNK)], SIDX[q],
                                  ISEM[q]).wait()
            pltpu.make_async_copy(ei_hbm.at[pl.ds(E + e0, CHUNK)], DIDX[q],
                                  ISEM[q]).wait()

        def gather_start(q, r):
            pltpu.async_copy(feat_hbm.at[SIDX[q]], ROWS[r], GSEM[r])

        def gather_wait(q, r):
            pltpu.make_async_copy(feat_hbm.at[SIDX[q]], ROWS[r],
                                  GSEM[r]).wait()

        def scatter_start(q, r):
            pltpu.async_copy(ROWS[r], acc.at[DIDX[q]], SSEM[r], add=True)

        def scatter_wait(q, r):
            pltpu.make_async_copy(ROWS[r], acc.at[DIDX[q]], SSEM[r]).wait()

        idx_start(0, 0)
        idx_start(1, 1)
        idx_wait(0, 0)
        gather_start(0, 0)
        idx_wait(1, 1)
        gather_start(1, 1)
        idx_start(2, 2)
        gather_wait(0, 0)
        scatter_start(0, 0)
        scatter_wait(0, 0)
        idx_wait(2, 2)
        gather_start(2, 0)
        idx_start(3, 3)
        gather_wait(1, 1)
        scatter_start(1, 1)

        @pl.loop(2, CHUNKS_PER_W, step=4)
        def _(t):
            for st in range(4):
                q = (2 + st) % 4
                r = (2 + st) % 2
                qn = (q + 1) % 4
                qp = (q + 2) % 4
                qm = (q + 3) % 4
                rn = (r + 1) % 2
                u = t + st
                scatter_wait(qm, rn)
                idx_wait(u + 1, qn)
                gather_start(qn, rn)
                idx_start(u + 2, qp)
                gather_wait(q, r)
                scatter_start(q, r)

        scatter_wait(1, 1)
        gather_wait(2, 0)
        idx_wait(CHUNKS_PER_W + 1, 3)

        @pl.when(wid < REM // 2)
        def _():
            e0 = (NW * CHUNKS_PER_W + wid * 2) * CHUNK
            pltpu.sync_copy(ei_hbm.at[pl.ds(e0, CHUNK)], sidx0)
            pltpu.sync_copy(ei_hbm.at[pl.ds(E + e0, CHUNK)], didx0)
            d0 = pltpu.async_copy(feat_hbm.at[sidx0], rows0, g0)
            pltpu.sync_copy(ei_hbm.at[pl.ds(e0 + CHUNK, CHUNK)], sidx1)
            pltpu.sync_copy(ei_hbm.at[pl.ds(E + e0 + CHUNK, CHUNK)], didx1)
            d1 = pltpu.async_copy(feat_hbm.at[sidx1], rows1, g1)
            d0.wait()
            pltpu.sync_copy(rows0, acc.at[didx0], add=True)
            d1.wait()
            pltpu.sync_copy(rows1, acc.at[didx1], add=True)

        plsc.subcore_barrier()

        @pl.when(s < NS - 1)
        def _():
            pltpu.sync_copy(acc.at[pl.ds(base_r, RPS)],
                            out_hbm.at[pl.ds(c * N + base_r, RPS)])

        @pl.when(s == NS - 1)
        def _():
            pltpu.sync_copy(acc.at[pl.ds(base_r, RPS_LAST)],
                            out_hbm.at[pl.ds(c * N + base_r, RPS_LAST)])

    return k(feat, ei_flat)


BR = 1000


def _tc_matmul(x, W, b2d):
    def body(x_ref, w_ref, b_ref, o_ref):
        o_ref[...] = jnp.dot(
            x_ref[...], w_ref[...], preferred_element_type=jnp.float32,
            precision=lax.Precision.HIGHEST) + b_ref[...]

    return pl.pallas_call(
        body,
        grid=(N // BR,),
        in_specs=[
            pl.BlockSpec((BR, D), lambda i: (i, 0)),
            pl.BlockSpec((D, D), lambda i: (0, 0)),
            pl.BlockSpec((1, D), lambda i: (0, 0)),
        ],
        out_specs=pl.BlockSpec((BR, D), lambda i: (i, 0)),
        out_shape=jax.ShapeDtypeStruct((N, D), jnp.float32),
    )(x, W, b2d)


def _tc_combine(z, p, W, final):
    def body(z_ref, p0_ref, p1_ref, w_ref, o_ref):
        t = p0_ref[...] + p1_ref[...]
        acc = jnp.dot(t, w_ref[...], preferred_element_type=jnp.float32,
                      precision=lax.Precision.HIGHEST) + z_ref[...]
        if final:
            m = jnp.max(acc, axis=1, keepdims=True)
            e = acc - m
            lse = jnp.log(jnp.sum(jnp.exp(e), axis=1, keepdims=True))
            o_ref[...] = e - lse
        else:
            o_ref[...] = jnp.maximum(acc, 0.0)

    nb = N // BR
    return pl.pallas_call(
        body,
        grid=(nb,),
        in_specs=[
            pl.BlockSpec((BR, D), lambda i: (i, 0)),
            pl.BlockSpec((BR, D), lambda i: (i, 0)),
            pl.BlockSpec((BR, D), lambda i: (i + nb, 0)),
            pl.BlockSpec((D, D), lambda i: (0, 0)),
        ],
        out_specs=pl.BlockSpec((BR, D), lambda i: (i, 0)),
        out_shape=jax.ShapeDtypeStruct((N, D), jnp.float32),
    )(z, p, p, W)


def kernel(input_feature, edge_index, W1, b1, W2, b2):
    ei_flat = edge_index.reshape(2 * E)
    b1_2d = b1.reshape(1, D)
    b2_2d = b2.reshape(1, D)

    p = _sc_segment_sum(input_feature, ei_flat)
    z1 = _tc_matmul(input_feature, W1, b1_2d)
    h = _tc_combine(z1, p, W1, final=False)
    q = _sc_segment_sum(h, ei_flat)
    z2 = _tc_matmul(h, W2, b2_2d)
    return _tc_combine(z2, q, W2, final=True)


# speedup vs baseline: 1.1632x; 1.0662x over previous
import functools

import jax
import jax.numpy as jnp
from jax import lax
from jax.experimental import pallas as pl
from jax.experimental.pallas import tpu as pltpu
from jax.experimental.pallas import tpu_sc as plsc

N = 10000
E = 320000
D = 128

NC = 2
NS = 16
NW = NC * NS

CHUNK = 128
NCHUNKS = E // CHUNK
CHUNKS_PER_W = NCHUNKS // NW
REM = NCHUNKS - CHUNKS_PER_W * NW

RPS = 632
RPS_LAST = N - RPS * (NS - 1)


def _sc_segment_sum(feat, ei_flat):
    mesh = plsc.VectorSubcoreMesh(core_axis_name="c", subcore_axis_name="s")

    @functools.partial(
        pl.kernel,
        out_type=jax.ShapeDtypeStruct((NC * N, D), jnp.float32),
        mesh=mesh,
        scratch_types=(
            [pltpu.VMEM((CHUNK,), jnp.int32)] * 8 +
            [pltpu.VMEM((CHUNK, D), jnp.float32)] * 3 +
            [pltpu.VMEM_SHARED((N, D), jnp.float32)] +
            [pltpu.SemaphoreType.DMA] * 10
        ),
    )
    def k(feat_hbm, ei_hbm, out_hbm,
          sidx0, sidx1, sidx2, sidx3, didx0, didx1, didx2, didx3,
          rows0, rows1, rows2, acc,
          i0, i1, i2, i3, g0, g1, g2, s0, s1, s2):
        c = lax.axis_index("c")
        s = lax.axis_index("s")
        wid = c * NS + s

        @pl.loop(0, CHUNK)
        def _(i):
            @pl.loop(0, D, step=16)
            def _(j):
                rows0.at[i, pl.ds(j, 16)][...] = jnp.zeros((16,), jnp.float32)

        base_r = s * RPS

        def zero_rows(tail):
            @pl.loop(0, 4)
            def _(r):
                pltpu.sync_copy(rows0, acc.at[pl.ds(base_r + r * CHUNK, CHUNK)])
            pltpu.sync_copy(rows0.at[pl.ds(0, tail)],
                            acc.at[pl.ds(base_r + 4 * CHUNK, tail)])

        @pl.when(s < NS - 1)
        def _():
            zero_rows(RPS - 4 * CHUNK)

        @pl.when(s == NS - 1)
        def _():
            zero_rows(RPS_LAST - 4 * CHUNK)

        plsc.subcore_barrier()

        base_c = wid * CHUNKS_PER_W
        SIDX = (sidx0, sidx1, sidx2, sidx3)
        DIDX = (didx0, didx1, didx2, didx3)
        ISEM = (i0, i1, i2, i3)
        ROWS = (rows0, rows1, rows2)
        GSEM = (g0, g1, g2)
        SSEM = (s0, s1, s2)

        def idx_start(u, q):
            e0 = (base_c + u) * CHUNK
            pltpu.async_copy(ei_hbm.at[pl.ds(e0, CHUNK)], SIDX[q], ISEM[q])
            pltpu.async_copy(ei_hbm.at[pl.ds(E + e0, CHUNK)], DIDX[q], ISEM[q])

        def idx_wait(u, q):
            e0 = (base_c + u) * CHUNK
            pltpu.make_async_copy(ei_hbm.at[pl.ds(e0, CHUNK)], SIDX[q],
                                  ISEM[q]).wait()
            pltpu.make_async_copy(ei_hbm.at[pl.ds(E + e0, CHUNK)], DIDX[q],
                                  ISEM[q]).wait()

        def gather_start(q, r):
            pltpu.async_copy(feat_hbm.at[SIDX[q]], ROWS[r], GSEM[r])

        def gather_wait(q, r):
            pltpu.make_async_copy(feat_hbm.at[SIDX[q]], ROWS[r],
                                  GSEM[r]).wait()

        def scatter_start(q, r):
            pltpu.async_copy(ROWS[r], acc.at[DIDX[q]], SSEM[r], add=True)

        def scatter_wait(q, r):
            pltpu.make_async_copy(ROWS[r], acc.at[DIDX[q]], SSEM[r]).wait()

        def stage(u, uoff, first=False):
            if not first:
                scatter_wait((uoff - 2) % 4, (uoff - 2) % 3)
            idx_wait(u + 1, (uoff + 1) % 4)
            gather_start((uoff + 1) % 4, (uoff + 1) % 3)
            idx_start(u + 2, (uoff + 2) % 4)
            gather_wait(uoff % 4, uoff % 3)
            scatter_start(uoff % 4, uoff % 3)

        idx_start(0, 0)
        idx_start(1, 1)
        idx_wait(0, 0)
        gather_start(0, 0)
        stage(0, 0, first=True)
        stage(1, 1, first=True)

        @pl.loop(2, 74, step=12)
        def _(t):
            for st in range(12):
                stage(t + st, 2 + st)

        for u in range(74, 78):
            stage(u, u)

        scatter_wait(76 % 4, 76 % 3)
        scatter_wait(77 % 4, 77 % 3)
        gather_wait(78 % 4, 78 % 3)
        idx_wait(CHUNKS_PER_W + 1, 79 % 4)

        @pl.when(wid < REM // 2)
        def _():
            e0 = (NW * CHUNKS_PER_W + wid * 2) * CHUNK
            pltpu.sync_copy(ei_hbm.at[pl.ds(e0, CHUNK)], sidx0)
            pltpu.sync_copy(ei_hbm.at[pl.ds(E + e0, CHUNK)], didx0)
            d0 = pltpu.async_copy(feat_hbm.at[sidx0], rows0, g0)
            pltpu.sync_copy(ei_hbm.at[pl.ds(e0 + CHUNK, CHUNK)], sidx1)
            pltpu.sync_copy(ei_hbm.at[pl.ds(E + e0 + CHUNK, CHUNK)], didx1)
            d1 = pltpu.async_copy(feat_hbm.at[sidx1], rows1, g1)
            d0.wait()
            pltpu.sync_copy(rows0, acc.at[didx0], add=True)
            d1.wait()
            pltpu.sync_copy(rows1, acc.at[didx1], add=True)

        plsc.subcore_barrier()

        @pl.when(s < NS - 1)
        def _():
            pltpu.sync_copy(acc.at[pl.ds(base_r, RPS)],
                            out_hbm.at[pl.ds(c * N + base_r, RPS)])

        @pl.when(s == NS - 1)
        def _():
            pltpu.sync_copy(acc.at[pl.ds(base_r, RPS_LAST)],
                            out_hbm.at[pl.ds(c * N + base_r, RPS_LAST)])

    return k(feat, ei_flat)


BR = 1000


def _tc_matmul(x, W, b2d):
    def body(x_ref, w_ref, b_ref, o_ref):
        o_ref[...] = jnp.dot(
            x_ref[...], w_ref[...], preferred_element_type=jnp.float32,
            precision=lax.Precision.HIGHEST) + b_ref[...]

    return pl.pallas_call(
        body,
        grid=(N // BR,),
        in_specs=[
            pl.BlockSpec((BR, D), lambda i: (i, 0)),
            pl.BlockSpec((D, D), lambda i: (0, 0)),
            pl.BlockSpec((1, D), lambda i: (0, 0)),
        ],
        out_specs=pl.BlockSpec((BR, D), lambda i: (i, 0)),
        out_shape=jax.ShapeDtypeStruct((N, D), jnp.float32),
    )(x, W, b2d)


def _tc_combine(z, p, W, final):
    def body(z_ref, p0_ref, p1_ref, w_ref, o_ref):
        t = p0_ref[...] + p1_ref[...]
        acc = jnp.dot(t, w_ref[...], preferred_element_type=jnp.float32,
                      precision=lax.Precision.HIGHEST) + z_ref[...]
        if final:
            m = jnp.max(acc, axis=1, keepdims=True)
            e = acc - m
            lse = jnp.log(jnp.sum(jnp.exp(e), axis=1, keepdims=True))
            o_ref[...] = e - lse
        else:
            o_ref[...] = jnp.maximum(acc, 0.0)

    nb = N // BR
    return pl.pallas_call(
        body,
        grid=(nb,),
        in_specs=[
            pl.BlockSpec((BR, D), lambda i: (i, 0)),
            pl.BlockSpec((BR, D), lambda i: (i, 0)),
            pl.BlockSpec((BR, D), lambda i: (i + nb, 0)),
            pl.BlockSpec((D, D), lambda i: (0, 0)),
        ],
        out_specs=pl.BlockSpec((BR, D), lambda i: (i, 0)),
        out_shape=jax.ShapeDtypeStruct((N, D), jnp.float32),
    )(z, p, p, W)


def kernel(input_feature, edge_index, W1, b1, W2, b2):
    ei_flat = edge_index.reshape(2 * E)
    b1_2d = b1.reshape(1, D)
    b2_2d = b2.reshape(1, D)

    p = _sc_segment_sum(input_feature, ei_flat)
    z1 = _tc_matmul(input_feature, W1, b1_2d)
    h = _tc_combine(z1, p, W1, final=False)
    q = _sc_segment_sum(h, ei_flat)
    z2 = _tc_matmul(h, W2, b2_2d)
    return _tc_combine(z2, q, W2, final=True)


# speedup vs baseline: 1.1912x; 1.0241x over previous
import functools

import jax
import jax.numpy as jnp
from jax import lax
from jax.experimental import pallas as pl
from jax.experimental.pallas import tpu as pltpu
from jax.experimental.pallas import tpu_sc as plsc

N = 10000
E = 320000
D = 128

NC = 2
NS = 16
NW = NC * NS

CHUNK = 128
NCHUNKS = E // CHUNK
CHUNKS_PER_W = NCHUNKS // NW
REM = NCHUNKS - CHUNKS_PER_W * NW

RPS = 632
RPS_LAST = N - RPS * (NS - 1)


def _sc_segment_sum(feat, ei_flat):
    mesh = plsc.VectorSubcoreMesh(core_axis_name="c", subcore_axis_name="s")

    @functools.partial(
        pl.kernel,
        out_type=jax.ShapeDtypeStruct((NC * N, D), jnp.float32),
        mesh=mesh,
        scratch_types=(
            [pltpu.VMEM((CHUNK,), jnp.int32)] * 8 +
            [pltpu.VMEM((CHUNK, D), jnp.float32)] * 3 +
            [pltpu.VMEM_SHARED((N, D), jnp.float32)] +
            [pltpu.SemaphoreType.DMA] * 10
        ),
    )
    def k(feat_hbm, ei_hbm, out_hbm,
          sidx0, sidx1, sidx2, sidx3, didx0, didx1, didx2, didx3,
          rows0, rows1, rows2, acc,
          i0, i1, i2, i3, g0, g1, g2, s0, s1, s2):
        c = lax.axis_index("c")
        s = lax.axis_index("s")
        wid = c * NS + s

        @pl.loop(0, CHUNK)
        def _(i):
            @pl.loop(0, D, step=16)
            def _(j):
                rows0.at[i, pl.ds(j, 16)][...] = jnp.zeros((16,), jnp.float32)

        base_r = s * RPS

        def zero_rows(tail):
            @pl.loop(0, 4)
            def _(r):
                pltpu.sync_copy(rows0, acc.at[pl.ds(base_r + r * CHUNK, CHUNK)])
            pltpu.sync_copy(rows0.at[pl.ds(0, tail)],
                            acc.at[pl.ds(base_r + 4 * CHUNK, tail)])

        @pl.when(s < NS - 1)
        def _():
            zero_rows(RPS - 4 * CHUNK)

        @pl.when(s == NS - 1)
        def _():
            zero_rows(RPS_LAST - 4 * CHUNK)

        plsc.subcore_barrier()

        base_c = wid * CHUNKS_PER_W
        SIDX = (sidx0, sidx1, sidx2, sidx3)
        DIDX = (didx0, didx1, didx2, didx3)
        ISEM = (i0, i1, i2, i3)
        ROWS = (rows0, rows1, rows2)
        GSEM = (g0, g1, g2)
        SSEM = (s0, s1, s2)

        def idx_start(u, q):
            e0 = (base_c + u) * CHUNK
            pltpu.async_copy(ei_hbm.at[pl.ds(e0, CHUNK)], SIDX[q], ISEM[q])
            pltpu.async_copy(ei_hbm.at[pl.ds(E + e0, CHUNK)], DIDX[q], ISEM[q])

        def idx_wait(u, q):
            e0 = (base_c + u) * CHUNK
            pltpu.make_async_copy(ei_hbm.at[pl.ds(e0, CHUNK)], SIDX[q],
                                  ISEM[q]).wait()
            pltpu.make_async_copy(ei_hbm.at[pl.ds(E + e0, CHUNK)], DIDX[q],
                                  ISEM[q]).wait()

        def gather_start(q, r):
            pltpu.async_copy(feat_hbm.at[SIDX[q]], ROWS[r], GSEM[r])

        def gather_wait(q, r):
            pltpu.make_async_copy(feat_hbm.at[SIDX[q]], ROWS[r],
                                  GSEM[r]).wait()

        def scatter_start(q, r):
            pltpu.async_copy(ROWS[r], acc.at[DIDX[q]], SSEM[r], add=True)

        def scatter_wait(q, r):
            pltpu.make_async_copy(ROWS[r], acc.at[DIDX[q]], SSEM[r]).wait()

        def stage(u, uoff, first=False):
            if not first:
                scatter_wait((uoff - 2) % 4, (uoff - 2) % 3)
            idx_wait(u + 1, (uoff + 1) % 4)
            gather_start((uoff + 1) % 4, (uoff + 1) % 3)
            idx_start(u + 2, (uoff + 2) % 4)
            gather_wait(uoff % 4, uoff % 3)
            scatter_start(uoff % 4, uoff % 3)

        idx_start(0, 0)
        idx_start(1, 1)
        idx_wait(0, 0)
        gather_start(0, 0)
        stage(0, 0, first=True)
        stage(1, 1, first=True)

        @pl.loop(2, 74, step=12)
        def _(t):
            for st in range(12):
                stage(t + st, 2 + st)

        for u in range(74, 78):
            stage(u, u)

        scatter_wait(76 % 4, 76 % 3)
        scatter_wait(77 % 4, 77 % 3)
        gather_wait(78 % 4, 78 % 3)
        idx_wait(CHUNKS_PER_W + 1, 79 % 4)

        @pl.when(wid < REM // 2)
        def _():
            e0 = (NW * CHUNKS_PER_W + wid * 2) * CHUNK
            pltpu.sync_copy(ei_hbm.at[pl.ds(e0, CHUNK)], sidx0)
            pltpu.sync_copy(ei_hbm.at[pl.ds(E + e0, CHUNK)], didx0)
            d0 = pltpu.async_copy(feat_hbm.at[sidx0], rows0, g0)
            pltpu.sync_copy(ei_hbm.at[pl.ds(e0 + CHUNK, CHUNK)], sidx1)
            pltpu.sync_copy(ei_hbm.at[pl.ds(E + e0 + CHUNK, CHUNK)], didx1)
            d1 = pltpu.async_copy(feat_hbm.at[sidx1], rows1, g1)
            d0.wait()
            pltpu.sync_copy(rows0, acc.at[didx0], add=True)
            d1.wait()
            pltpu.sync_copy(rows1, acc.at[didx1], add=True)

        plsc.subcore_barrier()

        @pl.when(s < NS - 1)
        def _():
            pltpu.sync_copy(acc.at[pl.ds(base_r, RPS)],
                            out_hbm.at[pl.ds(c * N + base_r, RPS)])

        @pl.when(s == NS - 1)
        def _():
            pltpu.sync_copy(acc.at[pl.ds(base_r, RPS_LAST)],
                            out_hbm.at[pl.ds(c * N + base_r, RPS_LAST)])

    return k(feat, ei_flat)


BR = 1000


def _tc_matmul(x, W, b2d):
    def body(x_ref, w_ref, b_ref, o_ref):
        o_ref[...] = jnp.dot(
            x_ref[...], w_ref[...], preferred_element_type=jnp.float32,
            precision=lax.Precision.DEFAULT) + b_ref[...]

    return pl.pallas_call(
        body,
        grid=(N // BR,),
        in_specs=[
            pl.BlockSpec((BR, D), lambda i: (i, 0)),
            pl.BlockSpec((D, D), lambda i: (0, 0)),
            pl.BlockSpec((1, D), lambda i: (0, 0)),
        ],
        out_specs=pl.BlockSpec((BR, D), lambda i: (i, 0)),
        out_shape=jax.ShapeDtypeStruct((N, D), jnp.float32),
    )(x, W, b2d)


def _tc_combine(z, p, W, final):
    def body(z_ref, p0_ref, p1_ref, w_ref, o_ref):
        t = p0_ref[...] + p1_ref[...]
        acc = jnp.dot(t, w_ref[...], preferred_element_type=jnp.float32,
                      precision=lax.Precision.DEFAULT) + z_ref[...]
        if final:
            m = jnp.max(acc, axis=1, keepdims=True)
            e = acc - m
            lse = jnp.log(jnp.sum(jnp.exp(e), axis=1, keepdims=True))
            o_ref[...] = e - lse
        else:
            o_ref[...] = jnp.maximum(acc, 0.0)

    nb = N // BR
    return pl.pallas_call(
        body,
        grid=(nb,),
        in_specs=[
            pl.BlockSpec((BR, D), lambda i: (i, 0)),
            pl.BlockSpec((BR, D), lambda i: (i, 0)),
            pl.BlockSpec((BR, D), lambda i: (i + nb, 0)),
            pl.BlockSpec((D, D), lambda i: (0, 0)),
        ],
        out_specs=pl.BlockSpec((BR, D), lambda i: (i, 0)),
        out_shape=jax.ShapeDtypeStruct((N, D), jnp.float32),
    )(z, p, p, W)


def kernel(input_feature, edge_index, W1, b1, W2, b2):
    ei_flat = edge_index.reshape(2 * E)
    b1_2d = b1.reshape(1, D)
    b2_2d = b2.reshape(1, D)

    p = _sc_segment_sum(input_feature, ei_flat)
    z1 = _tc_matmul(input_feature, W1, b1_2d)
    h = _tc_combine(z1, p, W1, final=False)
    q = _sc_segment_sum(h, ei_flat)
    z2 = _tc_matmul(h, W2, b2_2d)
    return _tc_combine(z2, q, W2, final=True)


# speedup vs baseline: 1.2160x; 1.0208x over previous
import functools

import jax
import jax.numpy as jnp
from jax import lax
from jax.experimental import pallas as pl
from jax.experimental.pallas import tpu as pltpu
from jax.experimental.pallas import tpu_sc as plsc

N = 10000
E = 320000
D = 128

NC = 2
NS = 16
NW = NC * NS

CHUNK = 128
NCHUNKS = E // CHUNK
CHUNKS_PER_W = NCHUNKS // NW
REM = NCHUNKS - CHUNKS_PER_W * NW

RPS = 632
RPS_LAST = N - RPS * (NS - 1)


def _sc_segment_sum(feat, ei_flat):
    mesh = plsc.VectorSubcoreMesh(core_axis_name="c", subcore_axis_name="s")

    @functools.partial(
        pl.kernel,
        out_type=jax.ShapeDtypeStruct((NC * N, D), jnp.float32),
        mesh=mesh,
        scratch_types=(
            [pltpu.VMEM((CHUNK,), jnp.int32)] * 8 +
            [pltpu.VMEM((CHUNK, D), jnp.float32)] * 3 +
            [pltpu.VMEM_SHARED((N, D), jnp.float32)] +
            [pltpu.SemaphoreType.DMA] * 10
        ),
    )
    def k(feat_hbm, ei_hbm, out_hbm,
          sidx0, sidx1, sidx2, sidx3, didx0, didx1, didx2, didx3,
          rows0, rows1, rows2, acc,
          i0, i1, i2, i3, g0, g1, g2, s0, s1, s2):
        c = lax.axis_index("c")
        s = lax.axis_index("s")
        wid = c * NS + s

        @pl.loop(0, CHUNK)
        def _(i):
            @pl.loop(0, D, step=16)
            def _(j):
                rows0.at[i, pl.ds(j, 16)][...] = jnp.zeros((16,), jnp.float32)

        base_r = s * RPS

        def zero_rows(tail):
            @pl.loop(0, 4)
            def _(r):
                pltpu.sync_copy(rows0, acc.at[pl.ds(base_r + r * CHUNK, CHUNK)])
            pltpu.sync_copy(rows0.at[pl.ds(0, tail)],
                            acc.at[pl.ds(base_r + 4 * CHUNK, tail)])

        @pl.when(s < NS - 1)
        def _():
            zero_rows(RPS - 4 * CHUNK)

        @pl.when(s == NS - 1)
        def _():
            zero_rows(RPS_LAST - 4 * CHUNK)

        plsc.subcore_barrier()

        base_c = wid * CHUNKS_PER_W
        SIDX = (sidx0, sidx1, sidx2, sidx3)
        DIDX = (didx0, didx1, didx2, didx3)
        ISEM = (i0, i1, i2, i3)
        ROWS = (rows0, rows1, rows2)
        GSEM = (g0, g1, g2)
        SSEM = (s0, s1, s2)

        def idx_start(u, q):
            e0 = (base_c + u) * CHUNK
            pltpu.async_copy(ei_hbm.at[pl.ds(e0, CHUNK)], SIDX[q], ISEM[q])
            pltpu.async_copy(ei_hbm.at[pl.ds(E + e0, CHUNK)], DIDX[q], ISEM[q])

        def idx_wait(u, q):
            e0 = (base_c + u) * CHUNK
            pltpu.make_async_copy(ei_hbm.at[pl.ds(e0, CHUNK)], SIDX[q],
                                  ISEM[q]).wait()
            pltpu.make_async_copy(ei_hbm.at[pl.ds(E + e0, CHUNK)], DIDX[q],
                                  ISEM[q]).wait()

        def gather_start(q, r):
            pltpu.async_copy(feat_hbm.at[SIDX[q]], ROWS[r], GSEM[r])

        def gather_wait(q, r):
            pltpu.make_async_copy(feat_hbm.at[SIDX[q]], ROWS[r],
                                  GSEM[r]).wait()

        def scatter_start(q, r):
            pltpu.async_copy(ROWS[r], acc.at[DIDX[q]], SSEM[r], add=True)

        def scatter_wait(q, r):
            pltpu.make_async_copy(ROWS[r], acc.at[DIDX[q]], SSEM[r]).wait()

        def stage(u, uoff, first=False):
            if not first:
                scatter_wait((uoff - 2) % 4, (uoff - 2) % 3)
            idx_wait(u + 1, (uoff + 1) % 4)
            gather_start((uoff + 1) % 4, (uoff + 1) % 3)
            idx_start(u + 2, (uoff + 2) % 4)
            gather_wait(uoff % 4, uoff % 3)
            scatter_start(uoff % 4, uoff % 3)

        idx_start(0, 0)
        idx_start(1, 1)
        idx_wait(0, 0)
        gather_start(0, 0)
        stage(0, 0, first=True)
        stage(1, 1, first=True)

        @pl.loop(2, 74, step=12)
        def _(t):
            for st in range(12):
                stage(t + st, 2 + st)

        for u in range(74, 78):
            stage(u, u)

        scatter_wait(76 % 4, 76 % 3)
        scatter_wait(77 % 4, 77 % 3)
        gather_wait(78 % 4, 78 % 3)
        idx_wait(CHUNKS_PER_W + 1, 79 % 4)

        @pl.when(wid < REM // 2)
        def _():
            e0 = (NW * CHUNKS_PER_W + wid * 2) * CHUNK
            pltpu.sync_copy(ei_hbm.at[pl.ds(e0, CHUNK)], sidx0)
            pltpu.sync_copy(ei_hbm.at[pl.ds(E + e0, CHUNK)], didx0)
            d0 = pltpu.async_copy(feat_hbm.at[sidx0], rows0, g0)
            pltpu.sync_copy(ei_hbm.at[pl.ds(e0 + CHUNK, CHUNK)], sidx1)
            pltpu.sync_copy(ei_hbm.at[pl.ds(E + e0 + CHUNK, CHUNK)], didx1)
            d1 = pltpu.async_copy(feat_hbm.at[sidx1], rows1, g1)
            d0.wait()
            pltpu.sync_copy(rows0, acc.at[didx0], add=True)
            d1.wait()
            pltpu.sync_copy(rows1, acc.at[didx1], add=True)

        plsc.subcore_barrier()

        @pl.when(s < NS - 1)
        def _():
            pltpu.sync_copy(acc.at[pl.ds(base_r, RPS)],
                            out_hbm.at[pl.ds(c * N + base_r, RPS)])

        @pl.when(s == NS - 1)
        def _():
            pltpu.sync_copy(acc.at[pl.ds(base_r, RPS_LAST)],
                            out_hbm.at[pl.ds(c * N + base_r, RPS_LAST)])

    return k(feat, ei_flat)


BR = 2000


def _tc_matmul(x, W, b2d):
    def body(x_ref, w_ref, b_ref, o_ref):
        o_ref[...] = jnp.dot(
            x_ref[...], w_ref[...], preferred_element_type=jnp.float32,
            precision=lax.Precision.DEFAULT) + b_ref[...]

    return pl.pallas_call(
        body,
        grid=(N // BR,),
        in_specs=[
            pl.BlockSpec((BR, D), lambda i: (i, 0)),
            pl.BlockSpec((D, D), lambda i: (0, 0)),
            pl.BlockSpec((1, D), lambda i: (0, 0)),
        ],
        out_specs=pl.BlockSpec((BR, D), lambda i: (i, 0)),
        out_shape=jax.ShapeDtypeStruct((N, D), jnp.float32),
    )(x, W, b2d)


def _tc_combine(z, p, W, final):
    def body(z_ref, p0_ref, p1_ref, w_ref, o_ref):
        t = p0_ref[...] + p1_ref[...]
        acc = jnp.dot(t, w_ref[...], preferred_element_type=jnp.float32,
                      precision=lax.Precision.DEFAULT) + z_ref[...]
        if final:
            m = jnp.max(acc, axis=1, keepdims=True)
            e = acc - m
            lse = jnp.log(jnp.sum(jnp.exp(e), axis=1, keepdims=True))
            o_ref[...] = e - lse
        else:
            o_ref[...] = jnp.maximum(acc, 0.0)

    nb = N // BR
    return pl.pallas_call(
        body,
        grid=(nb,),
        in_specs=[
            pl.BlockSpec((BR, D), lambda i: (i, 0)),
            pl.BlockSpec((BR, D), lambda i: (i, 0)),
            pl.BlockSpec((BR, D), lambda i: (i + nb, 0)),
            pl.BlockSpec((D, D), lambda i: (0, 0)),
        ],
        out_specs=pl.BlockSpec((BR, D), lambda i: (i, 0)),
        out_shape=jax.ShapeDtypeStruct((N, D), jnp.float32),
    )(z, p, p, W)


def kernel(input_feature, edge_index, W1, b1, W2, b2):
    ei_flat = edge_index.reshape(2 * E)
    b1_2d = b1.reshape(1, D)
    b2_2d = b2.reshape(1, D)

    p = _sc_segment_sum(input_feature, ei_flat)
    z1 = _tc_matmul(input_feature, W1, b1_2d)
    h = _tc_combine(z1, p, W1, final=False)
    q = _sc_segment_sum(h, ei_flat)
    z2 = _tc_matmul(h, W2, b2_2d)
    return _tc_combine(z2, q, W2, final=True)


# speedup vs baseline: 1.2272x; 1.0093x over previous
import functools

import jax
import jax.numpy as jnp
from jax import lax
from jax.experimental import pallas as pl
from jax.experimental.pallas import tpu as pltpu
from jax.experimental.pallas import tpu_sc as plsc

N = 10000
E = 320000
D = 128

NC = 2
NS = 16
NW = NC * NS

CHUNK = 128
NCHUNKS = E // CHUNK
CHUNKS_PER_W = NCHUNKS // NW
REM = NCHUNKS - CHUNKS_PER_W * NW

RPS = 632
RPS_LAST = N - RPS * (NS - 1)


def _sc_segment_sum(feat, ei_flat):
    mesh = plsc.VectorSubcoreMesh(core_axis_name="c", subcore_axis_name="s")

    @functools.partial(
        pl.kernel,
        out_type=jax.ShapeDtypeStruct((NC * N, D), jnp.float32),
        mesh=mesh,
        scratch_types=(
            [pltpu.VMEM((CHUNK,), jnp.int32)] * 8 +
            [pltpu.VMEM((CHUNK, D), jnp.float32)] * 3 +
            [pltpu.VMEM_SHARED((N, D), jnp.float32)] +
            [pltpu.SemaphoreType.DMA] * 10
        ),
    )
    def k(feat_hbm, ei_hbm, out_hbm,
          sidx0, sidx1, sidx2, sidx3, didx0, didx1, didx2, didx3,
          rows0, rows1, rows2, acc,
          i0, i1, i2, i3, g0, g1, g2, s0, s1, s2):
        c = lax.axis_index("c")
        s = lax.axis_index("s")
        wid = c * NS + s

        base_r = s * RPS
        base_c = wid * CHUNKS_PER_W
        SIDX = (sidx0, sidx1, sidx2, sidx3)
        DIDX = (didx0, didx1, didx2, didx3)
        ISEM = (i0, i1, i2, i3)
        ROWS = (rows0, rows1, rows2)
        GSEM = (g0, g1, g2)
        SSEM = (s0, s1, s2)

        def idx_start(u, q):
            e0 = (base_c + u) * CHUNK
            pltpu.async_copy(ei_hbm.at[pl.ds(e0, CHUNK)], SIDX[q], ISEM[q])
            pltpu.async_copy(ei_hbm.at[pl.ds(E + e0, CHUNK)], DIDX[q], ISEM[q])

        def idx_wait(u, q):
            e0 = (base_c + u) * CHUNK
            pltpu.make_async_copy(ei_hbm.at[pl.ds(e0, CHUNK)], SIDX[q],
                                  ISEM[q]).wait()
            pltpu.make_async_copy(ei_hbm.at[pl.ds(E + e0, CHUNK)], DIDX[q],
                                  ISEM[q]).wait()

        def gather_start(q, r):
            pltpu.async_copy(feat_hbm.at[SIDX[q]], ROWS[r], GSEM[r])

        def gather_wait(q, r):
            pltpu.make_async_copy(feat_hbm.at[SIDX[q]], ROWS[r],
                                  GSEM[r]).wait()

        def scatter_start(q, r):
            pltpu.async_copy(ROWS[r], acc.at[DIDX[q]], SSEM[r], add=True)

        def scatter_wait(q, r):
            pltpu.make_async_copy(ROWS[r], acc.at[DIDX[q]], SSEM[r]).wait()

        def stage(u, uoff, first=False):
            if not first:
                scatter_wait((uoff - 2) % 4, (uoff - 2) % 3)
            idx_wait(u + 1, (uoff + 1) % 4)
            gather_start((uoff + 1) % 4, (uoff + 1) % 3)
            idx_start(u + 2, (uoff + 2) % 4)
            gather_wait(uoff % 4, uoff % 3)
            scatter_start(uoff % 4, uoff % 3)

        idx_start(0, 0)
        idx_start(1, 1)

        @pl.loop(0, CHUNK)
        def _(i):
            @pl.loop(0, D, step=16)
            def _(j):
                rows0.at[i, pl.ds(j, 16)][...] = jnp.zeros((16,), jnp.float32)

        def zero_rows(tail):
            @pl.loop(0, 4)
            def _(r):
                pltpu.async_copy(rows0, acc.at[pl.ds(base_r + r * CHUNK,
                                                     CHUNK)], s2)
            pltpu.async_copy(rows0.at[pl.ds(0, tail)],
                             acc.at[pl.ds(base_r + 4 * CHUNK, tail)], s1)

            @pl.loop(0, 4)
            def _(r):
                pltpu.make_async_copy(rows0, acc.at[pl.ds(base_r + r * CHUNK,
                                                          CHUNK)], s2).wait()
            pltpu.make_async_copy(rows0.at[pl.ds(0, tail)],
                                  acc.at[pl.ds(base_r + 4 * CHUNK, tail)],
                                  s1).wait()

        @pl.when(s < NS - 1)
        def _():
            zero_rows(RPS - 4 * CHUNK)

        @pl.when(s == NS - 1)
        def _():
            zero_rows(RPS_LAST - 4 * CHUNK)

        idx_wait(0, 0)
        gather_start(0, 0)
        plsc.subcore_barrier()
        stage(0, 0, first=True)
        stage(1, 1, first=True)

        @pl.loop(2, 74, step=12)
        def _(t):
            for st in range(12):
                stage(t + st, 2 + st)

        for u in range(74, 78):
            stage(u, u)

        scatter_wait(76 % 4, 76 % 3)
        scatter_wait(77 % 4, 77 % 3)
        gather_wait(78 % 4, 78 % 3)
        idx_wait(CHUNKS_PER_W + 1, 79 % 4)

        @pl.when(wid < REM)
        def _():
            e0 = (NW * CHUNKS_PER_W + wid) * CHUNK
            pltpu.sync_copy(ei_hbm.at[pl.ds(e0, CHUNK)], sidx0)
            pltpu.sync_copy(ei_hbm.at[pl.ds(E + e0, CHUNK)], didx0)
            pltpu.sync_copy(feat_hbm.at[sidx0], rows0)
            pltpu.sync_copy(rows0, acc.at[didx0], add=True)

        plsc.subcore_barrier()

        @pl.when(s < NS - 1)
        def _():
            pltpu.sync_copy(acc.at[pl.ds(base_r, RPS)],
                            out_hbm.at[pl.ds(c * N + base_r, RPS)])

        @pl.when(s == NS - 1)
        def _():
            pltpu.sync_copy(acc.at[pl.ds(base_r, RPS_LAST)],
                            out_hbm.at[pl.ds(c * N + base_r, RPS_LAST)])

    return k(feat, ei_flat)


BR = 2000


def _tc_matmul(x, W, b2d):
    def body(x_ref, w_ref, b_ref, o_ref):
        o_ref[...] = jnp.dot(
            x_ref[...], w_ref[...], preferred_element_type=jnp.float32,
            precision=lax.Precision.DEFAULT) + b_ref[...]

    return pl.pallas_call(
        body,
        grid=(N // BR,),
        in_specs=[
            pl.BlockSpec((BR, D), lambda i: (i, 0)),
            pl.BlockSpec((D, D), lambda i: (0, 0)),
            pl.BlockSpec((1, D), lambda i: (0, 0)),
        ],
        out_specs=pl.BlockSpec((BR, D), lambda i: (i, 0)),
        out_shape=jax.ShapeDtypeStruct((N, D), jnp.float32),
    )(x, W, b2d)


def _tc_combine(z, p, W, final):
    def body(z_ref, p0_ref, p1_ref, w_ref, o_ref):
        t = p0_ref[...] + p1_ref[...]
        acc = jnp.dot(t, w_ref[...], preferred_element_type=jnp.float32,
                      precision=lax.Precision.DEFAULT) + z_ref[...]
        if final:
            m = jnp.max(acc, axis=1, keepdims=True)
            e = acc - m
            lse = jnp.log(jnp.sum(jnp.exp(e), axis=1, keepdims=True))
            o_ref[...] = e - lse
        else:
            o_ref[...] = jnp.maximum(acc, 0.0)

    nb = N // BR
    return pl.pallas_call(
        body,
        grid=(nb,),
        in_specs=[
            pl.BlockSpec((BR, D), lambda i: (i, 0)),
            pl.BlockSpec((BR, D), lambda i: (i, 0)),
            pl.BlockSpec((BR, D), lambda i: (i + nb, 0)),
            pl.BlockSpec((D, D), lambda i: (0, 0)),
        ],
        out_specs=pl.BlockSpec((BR, D), lambda i: (i, 0)),
        out_shape=jax.ShapeDtypeStruct((N, D), jnp.float32),
    )(z, p, p, W)


def kernel(input_feature, edge_index, W1, b1, W2, b2):
    ei_flat = edge_index.reshape(2 * E)
    b1_2d = b1.reshape(1, D)
    b2_2d = b2.reshape(1, D)

    p = _sc_segment_sum(input_feature, ei_flat)
    z1 = _tc_matmul(input_feature, W1, b1_2d)
    h = _tc_combine(z1, p, W1, final=False)
    q = _sc_segment_sum(h, ei_flat)
    z2 = _tc_matmul(h, W2, b2_2d)
    return _tc_combine(z2, q, W2, final=True)
